# baseline (speedup 1.0000x reference)
import functools

import jax
import jax.numpy as jnp
from jax.experimental import pallas as pl
from jax.experimental.pallas import tpu as pltpu

_EPS = 1e-5


def _shift_lanes(x, k):
    return pltpu.roll(x, k % x.shape[1], 1)


def _conv_image(x, w_ref, m_ref, hm_ref, cat_ref, hw):
    cin = x.shape[0]
    cat_ref[0:cin] = _shift_lanes(x, 1).astype(jnp.bfloat16) * m_ref[0]
    cat_ref[cin:2 * cin] = x.astype(jnp.bfloat16)
    cat_ref[2 * cin:3 * cin] = _shift_lanes(x, -1).astype(jnp.bfloat16) * m_ref[1]
    xcat = cat_ref[...]

    a0 = jnp.dot(w_ref[1], xcat, preferred_element_type=jnp.float32)
    am = jnp.dot(w_ref[0], xcat, preferred_element_type=jnp.float32)
    ap = jnp.dot(w_ref[2], xcat, preferred_element_type=jnp.float32)

    h, w = hw
    p = h * w
    cout = a0.shape[0]
    return (a0
            + _shift_lanes(am, w) * jnp.broadcast_to(hm_ref[0], (cout, p))
            + _shift_lanes(ap, p - w) * jnp.broadcast_to(hm_ref[1], (cout, p)))


def _fused_kernel(x_ref, w2_ref, m_ref, hm_ref, pm_ref, g_ref, b_ref, o_ref,
                  cat_ref, yc_ref, acc_ref, sc_ref, w3_ref, *, hw, n, upi):
    t = pl.program_id(0)
    i = pl.program_id(1)
    h, w = hw
    p = h * w
    cout = o_ref.shape[2]

    @pl.when(t == 0)
    def _phase0():
        @pl.when(i == 0)
        def _init():
            acc_ref[...] = jnp.zeros_like(acc_ref)
            for kh in range(3):
                w3_ref[kh] = jnp.dot(
                    w2_ref[...], pm_ref[kh],
                    preferred_element_type=jnp.float32).astype(jnp.bfloat16)

        s1 = jnp.zeros((cout, 1), jnp.float32)
        s2 = jnp.zeros((cout, 1), jnp.float32)
        for u in range(upi):
            y = _conv_image(x_ref[0, u], w3_ref, m_ref, hm_ref,
                            cat_ref.at[u], hw)
            yc_ref[i * upi + u] = y.astype(jnp.bfloat16)
            s1 += jnp.sum(y, axis=1, keepdims=True)
            s2 += jnp.sum(y * y, axis=1, keepdims=True)
        acc_ref[:, 0:1] += s1
        acc_ref[:, 1:2] += s2

    @pl.when(t == 1)
    def _phase1():
        @pl.when(i == 0)
        def _finalize():
            m_total = jnp.float32(n * p)
            mean = acc_ref[:, 0:1] / m_total
            var = jnp.maximum(acc_ref[:, 1:2] / m_total - mean * mean, 0.0)
            inv = jax.lax.rsqrt(var + _EPS)
            g_col = jnp.transpose(g_ref[...])
            b_col = jnp.transpose(b_ref[...])
            scale = g_col * inv
            sc_ref[:, 0:1] = scale
            sc_ref[:, 1:2] = b_col - mean * scale

        for u in range(upi):
            y = yc_ref[i * upi + u].astype(jnp.float32)
            o_ref[0, u] = (y * jnp.broadcast_to(sc_ref[:, 0:1], (cout, p))
                           + jnp.broadcast_to(sc_ref[:, 1:2], (cout, p)))


def kernel(x, conv_w, gamma, beta):
    n, cin, h, w = x.shape
    cout = conv_w.shape[0]
    p = h * w
    upi = 2 if n % 2 == 0 else 1
    steps = n // upi

    x3 = x.reshape(steps, upi, cin, p)
    w2 = conv_w.reshape(cout, cin * 9)
    g2 = gamma.astype(jnp.float32).reshape(1, cout)
    b2 = beta.astype(jnp.float32).reshape(1, cout)

    wcol = jnp.arange(p, dtype=jnp.int32) % w
    masks = jnp.stack([(wcol != 0), (wcol != w - 1)]).astype(jnp.bfloat16)
    masks = jnp.broadcast_to(masks[:, None, :], (2, cin, p))
    pix = jnp.arange(p, dtype=jnp.int32)
    hmasks = jnp.stack([(pix >= w), (pix < p - w)]).astype(
        jnp.float32).reshape(2, 1, p)
    li = jnp.arange(cin * 9, dtype=jnp.int32)[:, None]
    lj = jnp.arange(3 * cin, dtype=jnp.int32)[None, :]
    perm = jnp.stack([
        (li == (lj % cin) * 9 + kh * 3 + lj // cin).astype(jnp.float32)
        for kh in range(3)])

    out3 = pl.pallas_call(
        functools.partial(_fused_kernel, hw=(h, w), n=n, upi=upi),
        out_shape=jax.ShapeDtypeStruct((steps, upi, cout, p), x.dtype),
        grid=(2, steps),
        in_specs=[
            pl.BlockSpec((1, upi, cin, p), lambda t, i: ((1 - t) * i, 0, 0, 0)),
            pl.BlockSpec((cout, cin * 9), lambda t, i: (0, 0)),
            pl.BlockSpec((2, cin, p), lambda t, i: (0, 0, 0)),
            pl.BlockSpec((2, 1, p), lambda t, i: (0, 0, 0)),
            pl.BlockSpec((3, cin * 9, 3 * cin), lambda t, i: (0, 0, 0)),
            pl.BlockSpec((1, cout), lambda t, i: (0, 0)),
            pl.BlockSpec((1, cout), lambda t, i: (0, 0)),
        ],
        out_specs=pl.BlockSpec((1, upi, cout, p),
                               lambda t, i: (t * i, 0, 0, 0)),
        scratch_shapes=[
            pltpu.VMEM((upi, 3 * cin, p), jnp.bfloat16),
            pltpu.VMEM((n, cout, p), jnp.bfloat16),
            pltpu.VMEM((cout, 128), jnp.float32),
            pltpu.VMEM((cout, 128), jnp.float32),
            pltpu.VMEM((3, cout, 3 * cin), jnp.bfloat16),
        ],
        compiler_params=pltpu.CompilerParams(
            dimension_semantics=("arbitrary", "arbitrary"),
            vmem_limit_bytes=56 * 1024 * 1024,
            allow_input_fusion=[True] * 7,
        ),
    )(x3, w2, masks, hmasks, perm, g2, b2)

    return out3.reshape(n, cout, h, w)


# speedup vs baseline: 1.0821x; 1.0821x over previous
import functools

import numpy as np

import jax
import jax.numpy as jnp
from jax.experimental import pallas as pl
from jax.experimental.pallas import tpu as pltpu

_EPS = 1e-5


def _shift_lanes(x, k):
    return pltpu.roll(x, k % x.shape[1], 1)


def _conv_image(x, w_ref, m_ref, cat_ref, hw):
    cin = x.shape[0]
    cat_ref[0:cin] = _shift_lanes(x, 1).astype(jnp.bfloat16) * m_ref[0]
    cat_ref[cin:2 * cin] = x.astype(jnp.bfloat16)
    cat_ref[2 * cin:3 * cin] = _shift_lanes(x, -1).astype(jnp.bfloat16) * m_ref[1]
    xcat = cat_ref[...]

    a0 = jnp.dot(w_ref[1], xcat, preferred_element_type=jnp.float32)
    am = jnp.dot(w_ref[0], xcat, preferred_element_type=jnp.float32)
    ap = jnp.dot(w_ref[2], xcat, preferred_element_type=jnp.float32)

    h, w = hw
    p = h * w
    cout = a0.shape[0]
    z = jnp.zeros((cout, w), jnp.float32)
    return (a0
            + jnp.concatenate([z, am[:, :p - w]], axis=1)
            + jnp.concatenate([ap[:, w:], z], axis=1))


def _fused_kernel(x_ref, w2_ref, m_ref, pm_ref, g_ref, b_ref, o_ref,
                  cat_ref, yc_ref, acc_ref, sc_ref, w3_ref, *, hw, n, upi):
    t = pl.program_id(0)
    i = pl.program_id(1)
    h, w = hw
    p = h * w
    cout = o_ref.shape[2]

    @pl.when(t == 0)
    def _phase0():
        @pl.when(i == 0)
        def _init():
            acc_ref[...] = jnp.zeros_like(acc_ref)
            for kh in range(3):
                w3_ref[kh] = jnp.dot(
                    w2_ref[...], pm_ref[kh],
                    preferred_element_type=jnp.float32).astype(jnp.bfloat16)

        s1 = jnp.zeros((cout, 1), jnp.float32)
        s2 = jnp.zeros((cout, 1), jnp.float32)
        for u in range(upi):
            y = _conv_image(x_ref[0, u], w3_ref, m_ref, cat_ref.at[u], hw)
            yc_ref[i * upi + u] = y.astype(jnp.bfloat16)
            s1 += jnp.sum(y, axis=1, keepdims=True)
            s2 += jnp.sum(y * y, axis=1, keepdims=True)
        acc_ref[:, 0:1] += s1
        acc_ref[:, 1:2] += s2

    @pl.when(t == 1)
    def _phase1():
        @pl.when(i == 0)
        def _finalize():
            m_total = jnp.float32(n * p)
            mean = acc_ref[:, 0:1] / m_total
            var = jnp.maximum(acc_ref[:, 1:2] / m_total - mean * mean, 0.0)
            inv = jax.lax.rsqrt(var + _EPS)
            g_col = jnp.transpose(g_ref[...])
            b_col = jnp.transpose(b_ref[...])
            scale = g_col * inv
            sc_ref[:, 0:1] = scale
            sc_ref[:, 1:2] = b_col - mean * scale

        for u in range(upi):
            y = yc_ref[i * upi + u].astype(jnp.float32)
            o_ref[0, u] = (y * jnp.broadcast_to(sc_ref[:, 0:1], (cout, p))
                           + jnp.broadcast_to(sc_ref[:, 1:2], (cout, p)))


def kernel(x, conv_w, gamma, beta):
    n, cin, h, w = x.shape
    cout = conv_w.shape[0]
    p = h * w
    upi = 2 if n % 2 == 0 else 1
    steps = n // upi

    x3 = x.reshape(steps, upi, cin, p)
    w2 = conv_w.reshape(cout, cin * 9)
    g2 = gamma.astype(jnp.float32).reshape(1, cout)
    b2 = beta.astype(jnp.float32).reshape(1, cout)

    wcol = np.arange(p, dtype=np.int64) % w
    masks_np = np.stack([(wcol != 0), (wcol != w - 1)]).astype(np.float32)
    masks = jnp.asarray(
        np.broadcast_to(masks_np[:, None, :], (2, cin, p)),
        dtype=jnp.bfloat16)
    li = np.arange(cin * 9)[:, None]
    lj = np.arange(3 * cin)[None, :]
    perm = jnp.asarray(np.stack([
        (li == (lj % cin) * 9 + kh * 3 + lj // cin).astype(np.float32)
        for kh in range(3)]))

    out3 = pl.pallas_call(
        functools.partial(_fused_kernel, hw=(h, w), n=n, upi=upi),
        out_shape=jax.ShapeDtypeStruct((steps, upi, cout, p), x.dtype),
        grid=(2, steps),
        in_specs=[
            pl.BlockSpec((1, upi, cin, p), lambda t, i: ((1 - t) * i, 0, 0, 0)),
            pl.BlockSpec((cout, cin * 9), lambda t, i: (0, 0)),
            pl.BlockSpec((2, cin, p), lambda t, i: (0, 0, 0)),
            pl.BlockSpec((3, cin * 9, 3 * cin), lambda t, i: (0, 0, 0)),
            pl.BlockSpec((1, cout), lambda t, i: (0, 0)),
            pl.BlockSpec((1, cout), lambda t, i: (0, 0)),
        ],
        out_specs=pl.BlockSpec((1, upi, cout, p),
                               lambda t, i: (t * i, 0, 0, 0)),
        scratch_shapes=[
            pltpu.VMEM((upi, 3 * cin, p), jnp.bfloat16),
            pltpu.VMEM((n, cout, p), jnp.bfloat16),
            pltpu.VMEM((cout, 128), jnp.float32),
            pltpu.VMEM((cout, 128), jnp.float32),
            pltpu.VMEM((3, cout, 3 * cin), jnp.bfloat16),
        ],
        compiler_params=pltpu.CompilerParams(
            dimension_semantics=("arbitrary", "arbitrary"),
            vmem_limit_bytes=56 * 1024 * 1024,
            allow_input_fusion=[True] * 6,
        ),
    )(x3, w2, masks, perm, g2, b2)

    return out3.reshape(n, cout, h, w)


# speedup vs baseline: 1.1011x; 1.0175x over previous
import functools

import numpy as np

import jax
import jax.numpy as jnp
from jax.experimental import pallas as pl
from jax.experimental.pallas import tpu as pltpu

_EPS = 1e-5


def _shift_lanes(x, k):
    return pltpu.roll(x, k % x.shape[1], 1)


def _conv_image(x, w_ref, m_ref, cat_ref, hw):
    cin = x.shape[0]
    cat_ref[0:cin] = _shift_lanes(x, 1).astype(jnp.bfloat16) * m_ref[0]
    cat_ref[cin:2 * cin] = x.astype(jnp.bfloat16)
    cat_ref[2 * cin:3 * cin] = _shift_lanes(x, -1).astype(jnp.bfloat16) * m_ref[1]
    xcat = cat_ref[...]

    a0 = jnp.dot(w_ref[1], xcat, preferred_element_type=jnp.float32)
    am = jnp.dot(w_ref[0], xcat, preferred_element_type=jnp.float32)
    ap = jnp.dot(w_ref[2], xcat, preferred_element_type=jnp.float32)

    h, w = hw
    p = h * w
    cout = a0.shape[0]
    z = jnp.zeros((cout, w), jnp.float32)
    return (a0
            + jnp.concatenate([z, am[:, :p - w]], axis=1)
            + jnp.concatenate([ap[:, w:], z], axis=1))


def _fused_kernel(x_ref, w2_ref, m_ref, pm_ref, g_ref, b_ref, o_ref,
                  cat_ref, yc_ref, acc_ref, sc_ref, w3_ref, *, hw, n, upi):
    t = pl.program_id(0)
    i = pl.program_id(1)
    h, w = hw
    p = h * w
    cout = o_ref.shape[2]

    @pl.when(t == 0)
    def _phase0():
        @pl.when(i == 0)
        def _init():
            acc_ref[...] = jnp.zeros_like(acc_ref)
            for kh in range(3):
                w3_ref[kh] = jnp.dot(
                    w2_ref[...], pm_ref[kh],
                    preferred_element_type=jnp.float32).astype(jnp.bfloat16)

        s1 = jnp.zeros((cout, 1), jnp.float32)
        s2 = jnp.zeros((cout, 1), jnp.float32)
        for u in range(upi):
            y = _conv_image(x_ref[0, u], w3_ref, m_ref, cat_ref.at[u], hw)
            yc_ref[i * upi + u] = y.astype(jnp.bfloat16)
            s1 += jnp.sum(y, axis=1, keepdims=True)
            s2 += jnp.sum(y * y, axis=1, keepdims=True)
        acc_ref[:, 0:1] += s1
        acc_ref[:, 1:2] += s2

    @pl.when(t == 1)
    def _phase1():
        @pl.when(i == 0)
        def _finalize():
            m_total = jnp.float32(n * p)
            mean = acc_ref[:, 0:1] / m_total
            var = jnp.maximum(acc_ref[:, 1:2] / m_total - mean * mean, 0.0)
            inv = jax.lax.rsqrt(var + _EPS)
            g_col = jnp.transpose(g_ref[...])
            b_col = jnp.transpose(b_ref[...])
            scale = g_col * inv
            sc_ref[:, 0:1] = scale
            sc_ref[:, 1:2] = b_col - mean * scale

        for u in range(upi):
            y = yc_ref[i * upi + u].astype(jnp.float32)
            o_ref[0, u] = (y * jnp.broadcast_to(sc_ref[:, 0:1], (cout, p))
                           + jnp.broadcast_to(sc_ref[:, 1:2], (cout, p)))


def kernel(x, conv_w, gamma, beta):
    n, cin, h, w = x.shape
    cout = conv_w.shape[0]
    p = h * w
    upi = 4 if n % 4 == 0 else (2 if n % 2 == 0 else 1)
    steps = n // upi

    x3 = x.reshape(steps, upi, cin, p)
    w2 = conv_w.reshape(cout, cin * 9)
    g2 = gamma.astype(jnp.float32).reshape(1, cout)
    b2 = beta.astype(jnp.float32).reshape(1, cout)

    wcol = np.arange(p, dtype=np.int64) % w
    masks_np = np.stack([(wcol != 0), (wcol != w - 1)]).astype(np.float32)
    masks = jnp.asarray(
        np.broadcast_to(masks_np[:, None, :], (2, cin, p)),
        dtype=jnp.bfloat16)
    li = np.arange(cin * 9)[:, None]
    lj = np.arange(3 * cin)[None, :]
    perm = jnp.asarray(np.stack([
        (li == (lj % cin) * 9 + kh * 3 + lj // cin).astype(np.float32)
        for kh in range(3)]))

    out3 = pl.pallas_call(
        functools.partial(_fused_kernel, hw=(h, w), n=n, upi=upi),
        out_shape=jax.ShapeDtypeStruct((steps, upi, cout, p), x.dtype),
        grid=(2, steps),
        in_specs=[
            pl.BlockSpec((1, upi, cin, p), lambda t, i: ((1 - t) * i, 0, 0, 0)),
            pl.BlockSpec((cout, cin * 9), lambda t, i: (0, 0)),
            pl.BlockSpec((2, cin, p), lambda t, i: (0, 0, 0)),
            pl.BlockSpec((3, cin * 9, 3 * cin), lambda t, i: (0, 0, 0)),
            pl.BlockSpec((1, cout), lambda t, i: (0, 0)),
            pl.BlockSpec((1, cout), lambda t, i: (0, 0)),
        ],
        out_specs=pl.BlockSpec((1, upi, cout, p),
                               lambda t, i: (t * i, 0, 0, 0)),
        scratch_shapes=[
            pltpu.VMEM((upi, 3 * cin, p), jnp.bfloat16),
            pltpu.VMEM((n, cout, p), jnp.bfloat16),
            pltpu.VMEM((cout, 128), jnp.float32),
            pltpu.VMEM((cout, 128), jnp.float32),
            pltpu.VMEM((3, cout, 3 * cin), jnp.bfloat16),
        ],
        compiler_params=pltpu.CompilerParams(
            dimension_semantics=("arbitrary", "arbitrary"),
            vmem_limit_bytes=58 * 1024 * 1024,
            allow_input_fusion=[True] * 6,
        ),
    )(x3, w2, masks, perm, g2, b2)

    return out3.reshape(n, cout, h, w)
